# baseline (speedup 1.0000x reference)
import functools

import jax
import jax.numpy as jnp
from jax import lax
from jax.experimental import pallas as pl
from jax.experimental.pallas import tpu as pltpu
from jax.experimental.pallas import tpu_sc as plsc

N_NODES = 10000
NP = 10112
RPT = NP // 16
E = 320000
NW = 32
CHUNK = 256
NCHUNK = 40
EPT = CHUNK * NCHUNK
EPAD = EPT * NW
NBUF = 2
NITER = NCHUNK // (2 * NBUF)

_MESH = plsc.VectorSubcoreMesh(core_axis_name="c", subcore_axis_name="s")


def _make_propagate(D, H):
    @functools.partial(
        pl.kernel,
        out_type=jax.ShapeDtypeStruct((2, H, NP, D), jnp.float32),
        mesh=_MESH,
        scratch_types=[
            pltpu.VMEM((NCHUNK, CHUNK), jnp.int32),
            pltpu.VMEM((NCHUNK, CHUNK), jnp.int32),
            pltpu.VMEM((2, NBUF, CHUNK, D), jnp.float32),
        ] + [pltpu.VMEM_SHARED((NP, D), jnp.float32)] * (H + 1) + [
            pltpu.SemaphoreType.DMA,
            pltpu.SemaphoreType.DMA,
            pltpu.SemaphoreType.DMA,
            pltpu.SemaphoreType.DMA,
            pltpu.SemaphoreType.DMA,
        ],
        compiler_params=pltpu.CompilerParams(use_tc_tiling_on_sc=False),
    )
    def prop(zs_hbm, src_hbm, dst_hbm, zeros_hbm, out_hbm,
             srcs_v, dsts_v, rows_v, *shs_and_sems):
        acc_shs = shs_and_sems[:H]
        zs_sh = shs_and_sems[H]
        sem_i, sg0, sg1, ss0, ss1 = shs_and_sems[H + 1:]
        c = lax.axis_index("c")
        s = lax.axis_index("s")
        wid = c * 16 + s
        base = s * RPT
        pltpu.async_copy(src_hbm.at[wid], srcs_v, sem_i)
        pltpu.async_copy(dst_hbm.at[wid], dsts_v, sem_i)
        pltpu.sync_copy(zs_hbm.at[0, pl.ds(base, RPT)],
                        zs_sh.at[pl.ds(base, RPT)])
        for h in range(H):
            pltpu.sync_copy(zeros_hbm.at[pl.ds(base, RPT)],
                            acc_shs[h].at[pl.ds(base, RPT)])
        pltpu.make_async_copy(src_hbm.at[wid], srcs_v, sem_i).wait()
        pltpu.make_async_copy(dst_hbm.at[wid], dsts_v, sem_i).wait()
        plsc.subcore_barrier()

        sg = (sg0, sg1)
        ss = (ss0, ss1)

        def make_loop(zs_sh, zs_hb, acc_sh):
            def src_ref(grp, p, b):
                tbl = zs_sh if p == 0 else zs_hb
                return tbl.at[srcs_v.at[grp * NBUF + b]]

            def fire_gathers(grp, p):
                for b in range(NBUF):
                    pltpu.async_copy(src_ref(grp, p, b),
                                     rows_v.at[p, b], sg[p])

            def drain_gathers(grp, p):
                for b in range(NBUF):
                    pltpu.make_async_copy(src_ref(grp, p, b),
                                          rows_v.at[p, b], sg[p]).wait()

            def fire_scatters(grp, p):
                for b in range(NBUF):
                    pltpu.async_copy(rows_v.at[p, b],
                                     acc_sh.at[dsts_v.at[grp * NBUF + b]],
                                     ss[p], add=True)

            def drain_scatters(grp, p):
                for b in range(NBUF):
                    pltpu.make_async_copy(
                        rows_v.at[p, b],
                        acc_sh.at[dsts_v.at[grp * NBUF + b]],
                        ss[p]).wait()

            def body(i, carry):
                g0 = i * 2
                g1 = g0 + 1
                drain_gathers(g0, 0)
                fire_scatters(g0, 0)
                drain_gathers(g1, 1)
                fire_scatters(g1, 1)
                drain_scatters(g0, 0)

                @pl.when(i < NITER - 1)
                def _():
                    fire_gathers(g0 + 2, 0)

                drain_scatters(g1, 1)

                @pl.when(i < NITER - 1)
                def _():
                    fire_gathers(g1 + 2, 1)

                return carry

            fire_gathers(0, 0)
            fire_gathers(1, 1)
            lax.fori_loop(0, NITER, body, 0)

        for h in range(H):
            make_loop(zs_sh, zs_hbm.at[h], acc_shs[h])
            plsc.subcore_barrier()
            if h < H - 1:
                pltpu.async_copy(acc_shs[h].at[pl.ds(base, RPT)],
                                 out_hbm.at[c, h, pl.ds(base, RPT)], sem_i)
                pltpu.sync_copy(zs_hbm.at[h + 1, pl.ds(base, RPT)],
                                zs_sh.at[pl.ds(base, RPT)])
                plsc.subcore_barrier()
            else:
                for g in range(H - 1):
                    pltpu.make_async_copy(
                        acc_shs[g].at[pl.ds(base, RPT)],
                        out_hbm.at[c, g, pl.ds(base, RPT)], sem_i).wait()
                pltpu.sync_copy(acc_shs[h].at[pl.ds(base, RPT)],
                                out_hbm.at[c, h, pl.ds(base, RPT)])

    return prop


_prop32 = _make_propagate(32, 2)
_prop16 = _make_propagate(16, 1)


@functools.partial(
    pl.kernel,
    out_type=jax.ShapeDtypeStruct((2, NP, 8), jnp.float32),
    mesh=_MESH,
    scratch_types=[
        pltpu.VMEM((CHUNK,), jnp.int32),
        pltpu.VMEM((CHUNK, 8), jnp.float32),
        pltpu.VMEM_SHARED((NP, 8), jnp.float32),
    ],
    compiler_params=pltpu.CompilerParams(use_tc_tiling_on_sc=False),
)
def _degree(dst_hbm, ones_hbm, zeros_hbm, out_hbm, dst_v, ones_v, acc_sh):
    c = lax.axis_index("c")
    s = lax.axis_index("s")
    wid = c * 16 + s
    base = s * RPT
    pltpu.sync_copy(ones_hbm, ones_v)
    pltpu.sync_copy(zeros_hbm.at[pl.ds(base, RPT)],
                    acc_sh.at[pl.ds(base, RPT)])
    plsc.subcore_barrier()

    def body(ci, carry):
        pltpu.sync_copy(dst_hbm.at[wid, ci], dst_v)
        pltpu.sync_copy(ones_v, acc_sh.at[dst_v], add=True)
        return carry

    lax.fori_loop(0, NCHUNK, body, 0)
    plsc.subcore_barrier()
    pltpu.sync_copy(acc_sh.at[pl.ds(base, RPT)],
                    out_hbm.at[c, pl.ds(base, RPT)])


def _tc_prologue(deg, x_pad, w0):
    def body(deg_ref, x_ref, w_ref, dis_ref, zs_ref):
        d = deg_ref[0, :, 0:1] + deg_ref[1, :, 0:1] + 1.0
        dis = lax.rsqrt(d)
        dis_ref[...] = dis
        zs = jnp.dot(x_ref[...], w_ref[...],
                     preferred_element_type=jnp.float32) * dis
        zs_ref[0] = zs[:, :32]
        zs_ref[1] = zs[:, 32:]

    return pl.pallas_call(
        body,
        out_shape=[jax.ShapeDtypeStruct((NP, 1), jnp.float32),
                   jax.ShapeDtypeStruct((2, NP, 32), jnp.float32)],
    )(deg, x_pad, w0)


def _tc_layer(p, zs, dis, b, w_next, last):
    def body(p_ref, zs_ref, dis_ref, b_ref, w_ref, out_ref):
        dis = dis_ref[...]
        t = jnp.concatenate(
            [p_ref[0, 0] + p_ref[1, 0] + zs_ref[0],
             p_ref[0, 1] + p_ref[1, 1] + zs_ref[1]], axis=1)
        t = t * dis + b_ref[...]
        h = jnp.maximum(t, 0.0)
        z = jnp.dot(h, w_ref[...], preferred_element_type=jnp.float32) * dis
        if last:
            out_ref[0] = z
        else:
            out_ref[0] = z[:, :32]
            out_ref[1] = z[:, 32:]

    d_next = 16 if last else 32
    return pl.pallas_call(
        body,
        out_shape=jax.ShapeDtypeStruct((1 if last else 2, NP, d_next),
                                       jnp.float32),
    )(p, zs, dis, b, w_next)


def _tc_final(p, zs, dis, b):
    def body(p_ref, zs_ref, dis_ref, b_ref, out_ref):
        t = (p_ref[0, 0] + p_ref[1, 0] + zs_ref[0]) * dis_ref[...]
        out_ref[...] = jax.nn.sigmoid(t + b_ref[...])

    return pl.pallas_call(
        body,
        out_shape=jax.ShapeDtypeStruct((NP, 16), jnp.float32),
    )(p, zs, dis, b)


def kernel(x, edge_index, batch,
           W_gcn_0, b_gcn_0, W_gcn_h1, b_gcn_h1, W_gcn_h2, b_gcn_h2,
           W_gcn_h3, b_gcn_h3, W_gcn_h4, b_gcn_h4, W_gcn_h5, b_gcn_h5,
           W_gcn_h6, b_gcn_h6, W_gcn_out, b_gcn_out):
    del batch
    src = edge_index[0].astype(jnp.int32)
    dst = edge_index[1].astype(jnp.int32)
    src2 = jnp.pad(src, (0, EPAD - E)).reshape(NW, NCHUNK, CHUNK)
    dst2 = jnp.pad(dst, (0, EPAD - E),
                   constant_values=NP - 1).reshape(NW, NCHUNK, CHUNK)
    x_pad = jnp.pad(x, ((0, NP - N_NODES), (0, 0)))

    z32 = jnp.zeros((NP, 32), jnp.float32)
    z16 = jnp.zeros((NP, 16), jnp.float32)
    z8 = jnp.zeros((NP, 8), jnp.float32)
    ones_c = jnp.ones((CHUNK, 8), jnp.float32)

    deg = _degree(dst2, ones_c, z8)
    dis, zs = _tc_prologue(deg, x_pad, W_gcn_0)

    ws = [W_gcn_h1, W_gcn_h2, W_gcn_h3, W_gcn_h4, W_gcn_h5, W_gcn_h6]
    bs = [b_gcn_0, b_gcn_h1, b_gcn_h2, b_gcn_h3, b_gcn_h4, b_gcn_h5,
          b_gcn_h6]
    w_out16 = jnp.pad(W_gcn_out, ((0, 0), (0, 10)))
    b_out16 = jnp.pad(b_gcn_out, (0, 10)).reshape(1, 16)

    for i in range(7):
        p = _prop32(zs, src2, dst2, z32)
        if i < 6:
            zs = _tc_layer(p, zs, dis, bs[i].reshape(1, 64), ws[i], False)
        else:
            zs = _tc_layer(p, zs, dis, bs[i].reshape(1, 64), w_out16, True)

    p = _prop16(zs, src2, dst2, z16)
    out = _tc_final(p, zs, dis, b_out16)
    return out[:N_NODES, :6]


# speedup vs baseline: 1.4508x; 1.4508x over previous
import functools

import jax
import jax.numpy as jnp
from jax import lax
from jax.experimental import pallas as pl
from jax.experimental.pallas import tpu as pltpu
from jax.experimental.pallas import tpu_sc as plsc

N_NODES = 10000
NP = 10112
RPT = NP // 16
E = 320000
NW = 32
CHUNK = 256
NCHUNK = 40
EPT = CHUNK * NCHUNK
EPAD = EPT * NW
NBUF = 2
NITER = NCHUNK // (2 * NBUF)

_MESH = plsc.VectorSubcoreMesh(core_axis_name="c", subcore_axis_name="s")


def _make_propagate(D, H):
    @functools.partial(
        pl.kernel,
        out_type=jax.ShapeDtypeStruct((2, H, NP, D), jnp.float32),
        mesh=_MESH,
        scratch_types=[
            pltpu.VMEM((NCHUNK, CHUNK), jnp.int32),
            pltpu.VMEM((NCHUNK, CHUNK), jnp.int32),
            pltpu.VMEM((2, NBUF, CHUNK, D), jnp.float32),
        ] + [pltpu.VMEM_SHARED((NP, D), jnp.float32)] * (H + 1) + [
            pltpu.SemaphoreType.DMA,
            pltpu.SemaphoreType.DMA,
            pltpu.SemaphoreType.DMA,
            pltpu.SemaphoreType.DMA,
            pltpu.SemaphoreType.DMA,
        ],
        compiler_params=pltpu.CompilerParams(use_tc_tiling_on_sc=False),
    )
    def prop(zs_hbm, src_hbm, dst_hbm, zeros_hbm, out_hbm,
             srcs_v, dsts_v, rows_v, *shs_and_sems):
        acc_shs = shs_and_sems[:H]
        zs_sh = shs_and_sems[H]
        sem_i, sg0, sg1, ss0, ss1 = shs_and_sems[H + 1:]
        c = lax.axis_index("c")
        s = lax.axis_index("s")
        wid = c * 16 + s
        base = s * RPT
        pltpu.async_copy(src_hbm.at[wid], srcs_v, sem_i)
        pltpu.async_copy(dst_hbm.at[wid], dsts_v, sem_i)
        pltpu.sync_copy(zs_hbm.at[0, pl.ds(base, RPT)],
                        zs_sh.at[pl.ds(base, RPT)])
        for h in range(H):
            pltpu.sync_copy(zeros_hbm.at[pl.ds(base, RPT)],
                            acc_shs[h].at[pl.ds(base, RPT)])
        pltpu.make_async_copy(src_hbm.at[wid], srcs_v, sem_i).wait()
        pltpu.make_async_copy(dst_hbm.at[wid], dsts_v, sem_i).wait()
        plsc.subcore_barrier()

        sg = (sg0, sg1)
        ss = (ss0, ss1)

        def make_loop(zs_sh, zs_hb, acc_sh):
            def src_ref(grp, p, b):
                return zs_sh.at[srcs_v.at[grp * NBUF + b]]

            def fire_gathers(grp, p):
                for b in range(NBUF):
                    pltpu.async_copy(src_ref(grp, p, b),
                                     rows_v.at[p, b], sg[p])

            def drain_gathers(grp, p):
                for b in range(NBUF):
                    pltpu.make_async_copy(src_ref(grp, p, b),
                                          rows_v.at[p, b], sg[p]).wait()

            def fire_scatters(grp, p):
                for b in range(NBUF):
                    pltpu.async_copy(rows_v.at[p, b],
                                     acc_sh.at[dsts_v.at[grp * NBUF + b]],
                                     ss[p], add=True)

            def drain_scatters(grp, p):
                for b in range(NBUF):
                    pltpu.make_async_copy(
                        rows_v.at[p, b],
                        acc_sh.at[dsts_v.at[grp * NBUF + b]],
                        ss[p]).wait()

            def body(i, carry):
                g0 = i * 2
                g1 = g0 + 1
                drain_gathers(g0, 0)
                fire_scatters(g0, 0)
                drain_gathers(g1, 1)
                fire_scatters(g1, 1)
                drain_scatters(g0, 0)

                @pl.when(i < NITER - 1)
                def _():
                    fire_gathers(g0 + 2, 0)

                drain_scatters(g1, 1)

                @pl.when(i < NITER - 1)
                def _():
                    fire_gathers(g1 + 2, 1)

                return carry

            fire_gathers(0, 0)
            fire_gathers(1, 1)
            lax.fori_loop(0, NITER, body, 0)

        for h in range(H):
            make_loop(zs_sh, zs_hbm.at[h], acc_shs[h])
            plsc.subcore_barrier()
            if h < H - 1:
                pltpu.async_copy(acc_shs[h].at[pl.ds(base, RPT)],
                                 out_hbm.at[c, h, pl.ds(base, RPT)], sem_i)
                pltpu.sync_copy(zs_hbm.at[h + 1, pl.ds(base, RPT)],
                                zs_sh.at[pl.ds(base, RPT)])
                plsc.subcore_barrier()
            else:
                for g in range(H - 1):
                    pltpu.make_async_copy(
                        acc_shs[g].at[pl.ds(base, RPT)],
                        out_hbm.at[c, g, pl.ds(base, RPT)], sem_i).wait()
                pltpu.sync_copy(acc_shs[h].at[pl.ds(base, RPT)],
                                out_hbm.at[c, h, pl.ds(base, RPT)])

    return prop


_prop32 = _make_propagate(32, 2)
_prop16 = _make_propagate(16, 1)


@functools.partial(
    pl.kernel,
    out_type=jax.ShapeDtypeStruct((2, NP, 8), jnp.float32),
    mesh=_MESH,
    scratch_types=[
        pltpu.VMEM((CHUNK,), jnp.int32),
        pltpu.VMEM((CHUNK, 8), jnp.float32),
        pltpu.VMEM_SHARED((NP, 8), jnp.float32),
    ],
    compiler_params=pltpu.CompilerParams(use_tc_tiling_on_sc=False),
)
def _degree(dst_hbm, ones_hbm, zeros_hbm, out_hbm, dst_v, ones_v, acc_sh):
    c = lax.axis_index("c")
    s = lax.axis_index("s")
    wid = c * 16 + s
    base = s * RPT
    pltpu.sync_copy(ones_hbm, ones_v)
    pltpu.sync_copy(zeros_hbm.at[pl.ds(base, RPT)],
                    acc_sh.at[pl.ds(base, RPT)])
    plsc.subcore_barrier()

    def body(ci, carry):
        pltpu.sync_copy(dst_hbm.at[wid, ci], dst_v)
        pltpu.sync_copy(ones_v, acc_sh.at[dst_v], add=True)
        return carry

    lax.fori_loop(0, NCHUNK, body, 0)
    plsc.subcore_barrier()
    pltpu.sync_copy(acc_sh.at[pl.ds(base, RPT)],
                    out_hbm.at[c, pl.ds(base, RPT)])


def _tc_prologue(deg, x_pad, w0):
    def body(deg_ref, x_ref, w_ref, dis_ref, zs_ref):
        d = deg_ref[0, :, 0:1] + deg_ref[1, :, 0:1] + 1.0
        dis = lax.rsqrt(d)
        dis_ref[...] = dis
        zs = jnp.dot(x_ref[...], w_ref[...],
                     preferred_element_type=jnp.float32) * dis
        zs_ref[0] = zs[:, :32]
        zs_ref[1] = zs[:, 32:]

    return pl.pallas_call(
        body,
        out_shape=[jax.ShapeDtypeStruct((NP, 1), jnp.float32),
                   jax.ShapeDtypeStruct((2, NP, 32), jnp.float32)],
    )(deg, x_pad, w0)


def _tc_layer(p, zs, dis, b, w_next, last):
    def body(p_ref, zs_ref, dis_ref, b_ref, w_ref, out_ref):
        dis = dis_ref[...]
        t = jnp.concatenate(
            [p_ref[0, 0] + p_ref[1, 0] + zs_ref[0],
             p_ref[0, 1] + p_ref[1, 1] + zs_ref[1]], axis=1)
        t = t * dis + b_ref[...]
        h = jnp.maximum(t, 0.0)
        z = jnp.dot(h, w_ref[...], preferred_element_type=jnp.float32) * dis
        if last:
            out_ref[0] = z
        else:
            out_ref[0] = z[:, :32]
            out_ref[1] = z[:, 32:]

    d_next = 16 if last else 32
    return pl.pallas_call(
        body,
        out_shape=jax.ShapeDtypeStruct((1 if last else 2, NP, d_next),
                                       jnp.float32),
    )(p, zs, dis, b, w_next)


def _tc_final(p, zs, dis, b):
    def body(p_ref, zs_ref, dis_ref, b_ref, out_ref):
        t = (p_ref[0, 0] + p_ref[1, 0] + zs_ref[0]) * dis_ref[...]
        out_ref[...] = jax.nn.sigmoid(t + b_ref[...])

    return pl.pallas_call(
        body,
        out_shape=jax.ShapeDtypeStruct((NP, 16), jnp.float32),
    )(p, zs, dis, b)


def kernel(x, edge_index, batch,
           W_gcn_0, b_gcn_0, W_gcn_h1, b_gcn_h1, W_gcn_h2, b_gcn_h2,
           W_gcn_h3, b_gcn_h3, W_gcn_h4, b_gcn_h4, W_gcn_h5, b_gcn_h5,
           W_gcn_h6, b_gcn_h6, W_gcn_out, b_gcn_out):
    del batch
    src = edge_index[0].astype(jnp.int32)
    dst = edge_index[1].astype(jnp.int32)
    src2 = jnp.pad(src, (0, EPAD - E)).reshape(NW, NCHUNK, CHUNK)
    dst2 = jnp.pad(dst, (0, EPAD - E),
                   constant_values=NP - 1).reshape(NW, NCHUNK, CHUNK)
    x_pad = jnp.pad(x, ((0, NP - N_NODES), (0, 0)))

    z32 = jnp.zeros((NP, 32), jnp.float32)
    z16 = jnp.zeros((NP, 16), jnp.float32)
    z8 = jnp.zeros((NP, 8), jnp.float32)
    ones_c = jnp.ones((CHUNK, 8), jnp.float32)

    deg = _degree(dst2, ones_c, z8)
    dis, zs = _tc_prologue(deg, x_pad, W_gcn_0)

    ws = [W_gcn_h1, W_gcn_h2, W_gcn_h3, W_gcn_h4, W_gcn_h5, W_gcn_h6]
    bs = [b_gcn_0, b_gcn_h1, b_gcn_h2, b_gcn_h3, b_gcn_h4, b_gcn_h5,
          b_gcn_h6]
    w_out16 = jnp.pad(W_gcn_out, ((0, 0), (0, 10)))
    b_out16 = jnp.pad(b_gcn_out, (0, 10)).reshape(1, 16)

    for i in range(7):
        p = _prop32(zs, src2, dst2, z32)
        if i < 6:
            zs = _tc_layer(p, zs, dis, bs[i].reshape(1, 64), ws[i], False)
        else:
            zs = _tc_layer(p, zs, dis, bs[i].reshape(1, 64), w_out16, True)

    p = _prop16(zs, src2, dst2, z16)
    out = _tc_final(p, zs, dis, b_out16)
    return out[:N_NODES, :6]


# speedup vs baseline: 1.4517x; 1.0007x over previous
import functools

import jax
import jax.numpy as jnp
from jax import lax
from jax.experimental import pallas as pl
from jax.experimental.pallas import tpu as pltpu
from jax.experimental.pallas import tpu_sc as plsc

N_NODES = 10000
NP = 10112
RPT = NP // 16
E = 320000
NW = 32
CHUNK = 256
NCHUNK = 40
EPT = CHUNK * NCHUNK
EPAD = EPT * NW
NBUF = 2
NITER = NCHUNK // (2 * NBUF)

_MESH = plsc.VectorSubcoreMesh(core_axis_name="c", subcore_axis_name="s")


def _make_propagate(D, H):
    @functools.partial(
        pl.kernel,
        out_type=jax.ShapeDtypeStruct((2, H, NP, D), jnp.float32),
        mesh=_MESH,
        scratch_types=[
            pltpu.VMEM((NCHUNK, CHUNK), jnp.int32),
            pltpu.VMEM((NCHUNK, CHUNK), jnp.int32),
            pltpu.VMEM((2, NBUF, CHUNK, D), jnp.float32),
        ] + [pltpu.VMEM_SHARED((NP, D), jnp.float32)] * (H + 1) + [
            pltpu.SemaphoreType.DMA,
            pltpu.SemaphoreType.DMA,
            pltpu.SemaphoreType.DMA,
            pltpu.SemaphoreType.DMA,
            pltpu.SemaphoreType.DMA,
        ],
        compiler_params=pltpu.CompilerParams(use_tc_tiling_on_sc=False,
                                             skip_device_barrier=True),
    )
    def prop(zs_hbm, src_hbm, dst_hbm, zeros_hbm, out_hbm,
             srcs_v, dsts_v, rows_v, *shs_and_sems):
        acc_shs = shs_and_sems[:H]
        zs_sh = shs_and_sems[H]
        sem_i, sg0, sg1, ss0, ss1 = shs_and_sems[H + 1:]
        c = lax.axis_index("c")
        s = lax.axis_index("s")
        wid = c * 16 + s
        base = s * RPT
        pltpu.async_copy(src_hbm.at[wid], srcs_v, sem_i)
        pltpu.async_copy(dst_hbm.at[wid], dsts_v, sem_i)
        pltpu.sync_copy(zs_hbm.at[0, pl.ds(base, RPT)],
                        zs_sh.at[pl.ds(base, RPT)])
        for h in range(H):
            pltpu.sync_copy(zeros_hbm.at[pl.ds(base, RPT)],
                            acc_shs[h].at[pl.ds(base, RPT)])
        pltpu.make_async_copy(src_hbm.at[wid], srcs_v, sem_i).wait()
        pltpu.make_async_copy(dst_hbm.at[wid], dsts_v, sem_i).wait()
        plsc.subcore_barrier()

        sg = (sg0, sg1)
        ss = (ss0, ss1)

        def make_loop(zs_sh, zs_hb, acc_sh):
            def src_ref(grp, p, b):
                return zs_sh.at[srcs_v.at[grp * NBUF + b]]

            def fire_gathers(grp, p):
                for b in range(NBUF):
                    pltpu.async_copy(src_ref(grp, p, b),
                                     rows_v.at[p, b], sg[p])

            def drain_gathers(grp, p):
                for b in range(NBUF):
                    pltpu.make_async_copy(src_ref(grp, p, b),
                                          rows_v.at[p, b], sg[p]).wait()

            def fire_scatters(grp, p):
                for b in range(NBUF):
                    pltpu.async_copy(rows_v.at[p, b],
                                     acc_sh.at[dsts_v.at[grp * NBUF + b]],
                                     ss[p], add=True)

            def drain_scatters(grp, p):
                for b in range(NBUF):
                    pltpu.make_async_copy(
                        rows_v.at[p, b],
                        acc_sh.at[dsts_v.at[grp * NBUF + b]],
                        ss[p]).wait()

            def body(i, carry):
                g0 = i * 2
                g1 = g0 + 1
                drain_gathers(g0, 0)
                fire_scatters(g0, 0)
                drain_gathers(g1, 1)
                fire_scatters(g1, 1)
                drain_scatters(g0, 0)

                @pl.when(i < NITER - 1)
                def _():
                    fire_gathers(g0 + 2, 0)

                drain_scatters(g1, 1)

                @pl.when(i < NITER - 1)
                def _():
                    fire_gathers(g1 + 2, 1)

                return carry

            fire_gathers(0, 0)
            fire_gathers(1, 1)
            lax.fori_loop(0, NITER, body, 0)

        for h in range(H):
            make_loop(zs_sh, zs_hbm.at[h], acc_shs[h])
            plsc.subcore_barrier()
            if h < H - 1:
                pltpu.async_copy(acc_shs[h].at[pl.ds(base, RPT)],
                                 out_hbm.at[c, h, pl.ds(base, RPT)], sem_i)
                pltpu.sync_copy(zs_hbm.at[h + 1, pl.ds(base, RPT)],
                                zs_sh.at[pl.ds(base, RPT)])
                plsc.subcore_barrier()
            else:
                for g in range(H - 1):
                    pltpu.make_async_copy(
                        acc_shs[g].at[pl.ds(base, RPT)],
                        out_hbm.at[c, g, pl.ds(base, RPT)], sem_i).wait()
                pltpu.sync_copy(acc_shs[h].at[pl.ds(base, RPT)],
                                out_hbm.at[c, h, pl.ds(base, RPT)])

    return prop


_prop32 = _make_propagate(32, 2)
_prop16 = _make_propagate(16, 1)


@functools.partial(
    pl.kernel,
    out_type=jax.ShapeDtypeStruct((2, NP, 8), jnp.float32),
    mesh=_MESH,
    scratch_types=[
        pltpu.VMEM((CHUNK,), jnp.int32),
        pltpu.VMEM((CHUNK, 8), jnp.float32),
        pltpu.VMEM_SHARED((NP, 8), jnp.float32),
    ],
    compiler_params=pltpu.CompilerParams(use_tc_tiling_on_sc=False,
                                         skip_device_barrier=True),
)
def _degree(dst_hbm, ones_hbm, zeros_hbm, out_hbm, dst_v, ones_v, acc_sh):
    c = lax.axis_index("c")
    s = lax.axis_index("s")
    wid = c * 16 + s
    base = s * RPT
    pltpu.sync_copy(ones_hbm, ones_v)
    pltpu.sync_copy(zeros_hbm.at[pl.ds(base, RPT)],
                    acc_sh.at[pl.ds(base, RPT)])
    plsc.subcore_barrier()

    def body(ci, carry):
        pltpu.sync_copy(dst_hbm.at[wid, ci], dst_v)
        pltpu.sync_copy(ones_v, acc_sh.at[dst_v], add=True)
        return carry

    lax.fori_loop(0, NCHUNK, body, 0)
    plsc.subcore_barrier()
    pltpu.sync_copy(acc_sh.at[pl.ds(base, RPT)],
                    out_hbm.at[c, pl.ds(base, RPT)])


def _tc_prologue(deg, x_pad, w0):
    def body(deg_ref, x_ref, w_ref, dis_ref, zs_ref):
        d = deg_ref[0, :, 0:1] + deg_ref[1, :, 0:1] + 1.0
        dis = lax.rsqrt(d)
        dis_ref[...] = dis
        zs = jnp.dot(x_ref[...], w_ref[...],
                     preferred_element_type=jnp.float32) * dis
        zs_ref[0] = zs[:, :32]
        zs_ref[1] = zs[:, 32:]

    return pl.pallas_call(
        body,
        out_shape=[jax.ShapeDtypeStruct((NP, 1), jnp.float32),
                   jax.ShapeDtypeStruct((2, NP, 32), jnp.float32)],
    )(deg, x_pad, w0)


def _tc_layer(p, zs, dis, b, w_next, last):
    def body(p_ref, zs_ref, dis_ref, b_ref, w_ref, out_ref):
        dis = dis_ref[...]
        t = jnp.concatenate(
            [p_ref[0, 0] + p_ref[1, 0] + zs_ref[0],
             p_ref[0, 1] + p_ref[1, 1] + zs_ref[1]], axis=1)
        t = t * dis + b_ref[...]
        h = jnp.maximum(t, 0.0)
        z = jnp.dot(h, w_ref[...], preferred_element_type=jnp.float32) * dis
        if last:
            out_ref[0] = z
        else:
            out_ref[0] = z[:, :32]
            out_ref[1] = z[:, 32:]

    d_next = 16 if last else 32
    return pl.pallas_call(
        body,
        out_shape=jax.ShapeDtypeStruct((1 if last else 2, NP, d_next),
                                       jnp.float32),
    )(p, zs, dis, b, w_next)


def _tc_final(p, zs, dis, b):
    def body(p_ref, zs_ref, dis_ref, b_ref, out_ref):
        t = (p_ref[0, 0] + p_ref[1, 0] + zs_ref[0]) * dis_ref[...]
        out_ref[...] = jax.nn.sigmoid(t + b_ref[...])

    return pl.pallas_call(
        body,
        out_shape=jax.ShapeDtypeStruct((NP, 16), jnp.float32),
    )(p, zs, dis, b)


def kernel(x, edge_index, batch,
           W_gcn_0, b_gcn_0, W_gcn_h1, b_gcn_h1, W_gcn_h2, b_gcn_h2,
           W_gcn_h3, b_gcn_h3, W_gcn_h4, b_gcn_h4, W_gcn_h5, b_gcn_h5,
           W_gcn_h6, b_gcn_h6, W_gcn_out, b_gcn_out):
    del batch
    src = edge_index[0].astype(jnp.int32)
    dst = edge_index[1].astype(jnp.int32)
    src2 = jnp.pad(src, (0, EPAD - E)).reshape(NW, NCHUNK, CHUNK)
    dst2 = jnp.pad(dst, (0, EPAD - E),
                   constant_values=NP - 1).reshape(NW, NCHUNK, CHUNK)
    x_pad = jnp.pad(x, ((0, NP - N_NODES), (0, 0)))

    z32 = jnp.zeros((NP, 32), jnp.float32)
    z16 = jnp.zeros((NP, 16), jnp.float32)
    z8 = jnp.zeros((NP, 8), jnp.float32)
    ones_c = jnp.ones((CHUNK, 8), jnp.float32)

    deg = _degree(dst2, ones_c, z8)
    dis, zs = _tc_prologue(deg, x_pad, W_gcn_0)

    ws = [W_gcn_h1, W_gcn_h2, W_gcn_h3, W_gcn_h4, W_gcn_h5, W_gcn_h6]
    bs = [b_gcn_0, b_gcn_h1, b_gcn_h2, b_gcn_h3, b_gcn_h4, b_gcn_h5,
          b_gcn_h6]
    w_out16 = jnp.pad(W_gcn_out, ((0, 0), (0, 10)))
    b_out16 = jnp.pad(b_gcn_out, (0, 10)).reshape(1, 16)

    for i in range(7):
        p = _prop32(zs, src2, dst2, z32)
        if i < 6:
            zs = _tc_layer(p, zs, dis, bs[i].reshape(1, 64), ws[i], False)
        else:
            zs = _tc_layer(p, zs, dis, bs[i].reshape(1, 64), w_out16, True)

    p = _prop16(zs, src2, dst2, z16)
    out = _tc_final(p, zs, dis, b_out16)
    return out[:N_NODES, :6]


# speedup vs baseline: 1.4709x; 1.0132x over previous
import functools

import jax
import jax.numpy as jnp
from jax import lax
from jax.experimental import pallas as pl
from jax.experimental.pallas import tpu as pltpu
from jax.experimental.pallas import tpu_sc as plsc

N_NODES = 10000
NP = 10112
RPT = NP // 16
E = 320000
NW = 32
CHUNK = 256
NCHUNK = 40
EPT = CHUNK * NCHUNK
EPAD = EPT * NW
NBUF = 2
NITER = NCHUNK // (2 * NBUF)

_MESH = plsc.VectorSubcoreMesh(core_axis_name="c", subcore_axis_name="s")


def _make_propagate(D, H):
    @functools.partial(
        pl.kernel,
        out_type=jax.ShapeDtypeStruct((2, H, NP, D), jnp.float32),
        mesh=_MESH,
        scratch_types=[
            pltpu.VMEM((NCHUNK, CHUNK), jnp.int32),
            pltpu.VMEM((NCHUNK, CHUNK), jnp.int32),
            pltpu.VMEM((2, NBUF, CHUNK, D), jnp.float32),
        ] + [pltpu.VMEM_SHARED((NP, D), jnp.float32)] * (H + 1) + [
            pltpu.SemaphoreType.DMA,
            pltpu.SemaphoreType.DMA,
            pltpu.SemaphoreType.DMA,
            pltpu.SemaphoreType.DMA,
            pltpu.SemaphoreType.DMA,
        ],
        compiler_params=pltpu.CompilerParams(use_tc_tiling_on_sc=False),
    )
    def prop(zs_hbm, src_hbm, dst_hbm, zeros_hbm, out_hbm,
             srcs_v, dsts_v, rows_v, *shs_and_sems):
        acc_shs = shs_and_sems[:H]
        zs_sh = shs_and_sems[H]
        sem_i, sg0, sg1, ss0, ss1 = shs_and_sems[H + 1:]
        c = lax.axis_index("c")
        s = lax.axis_index("s")
        wid = c * 16 + s
        base = s * RPT
        pltpu.async_copy(src_hbm.at[wid], srcs_v, sem_i)
        pltpu.async_copy(dst_hbm.at[wid], dsts_v, sem_i)
        pltpu.sync_copy(zs_hbm.at[0, pl.ds(base, RPT)],
                        zs_sh.at[pl.ds(base, RPT)])
        for h in range(H):
            pltpu.sync_copy(zeros_hbm.at[pl.ds(base, RPT)],
                            acc_shs[h].at[pl.ds(base, RPT)])
        pltpu.make_async_copy(src_hbm.at[wid], srcs_v, sem_i).wait()
        pltpu.make_async_copy(dst_hbm.at[wid], dsts_v, sem_i).wait()
        plsc.subcore_barrier()

        sg = (sg0, sg1)
        ss = (ss0, ss1)

        def make_loop(zs_sh, zs_hb, acc_sh):
            def src_ref(grp, p, b):
                return zs_sh.at[srcs_v.at[grp * NBUF + b]]

            def fire_gathers(grp, p):
                for b in range(NBUF):
                    pltpu.async_copy(src_ref(grp, p, b),
                                     rows_v.at[p, b], sg[p])

            def drain_gathers(grp, p):
                for b in range(NBUF):
                    pltpu.make_async_copy(src_ref(grp, p, b),
                                          rows_v.at[p, b], sg[p]).wait()

            def fire_scatters(grp, p):
                for b in range(NBUF):
                    pltpu.async_copy(rows_v.at[p, b],
                                     acc_sh.at[dsts_v.at[grp * NBUF + b]],
                                     ss[p], add=True)

            def drain_scatters(grp, p):
                for b in range(NBUF):
                    pltpu.make_async_copy(
                        rows_v.at[p, b],
                        acc_sh.at[dsts_v.at[grp * NBUF + b]],
                        ss[p]).wait()

            def body(i, carry):
                g0 = i * 2
                g1 = g0 + 1
                drain_gathers(g0, 0)
                fire_scatters(g0, 0)
                drain_gathers(g1, 1)
                fire_scatters(g1, 1)
                drain_scatters(g0, 0)

                @pl.when(i < NITER - 1)
                def _():
                    fire_gathers(g0 + 2, 0)

                drain_scatters(g1, 1)

                @pl.when(i < NITER - 1)
                def _():
                    fire_gathers(g1 + 2, 1)

                return carry

            fire_gathers(0, 0)
            fire_gathers(1, 1)
            lax.fori_loop(0, NITER, body, 0)

        for h in range(H):
            make_loop(zs_sh, zs_hbm.at[h], acc_shs[h])
            plsc.subcore_barrier()
            if h < H - 1:
                pltpu.async_copy(acc_shs[h].at[pl.ds(base, RPT)],
                                 out_hbm.at[c, h, pl.ds(base, RPT)], sem_i)
                pltpu.sync_copy(zs_hbm.at[h + 1, pl.ds(base, RPT)],
                                zs_sh.at[pl.ds(base, RPT)])
                plsc.subcore_barrier()
            else:
                for g in range(H - 1):
                    pltpu.make_async_copy(
                        acc_shs[g].at[pl.ds(base, RPT)],
                        out_hbm.at[c, g, pl.ds(base, RPT)], sem_i).wait()
                pltpu.sync_copy(acc_shs[h].at[pl.ds(base, RPT)],
                                out_hbm.at[c, h, pl.ds(base, RPT)])

    return prop


_prop32 = _make_propagate(32, 2)
_prop16 = _make_propagate(16, 1)


@functools.partial(
    pl.kernel,
    out_type=jax.ShapeDtypeStruct((2, NP, 8), jnp.float32),
    mesh=_MESH,
    scratch_types=[
        pltpu.VMEM((NCHUNK, CHUNK), jnp.int32),
        pltpu.VMEM((CHUNK, 8), jnp.float32),
        pltpu.VMEM_SHARED((NP, 8), jnp.float32),
        pltpu.SemaphoreType.DMA,
        pltpu.SemaphoreType.DMA,
    ],
    compiler_params=pltpu.CompilerParams(use_tc_tiling_on_sc=False),
)
def _degree(dst_hbm, ones_hbm, zeros_hbm, out_hbm, dsts_v, ones_v, acc_sh,
            sem_i, sem_s):
    c = lax.axis_index("c")
    s = lax.axis_index("s")
    wid = c * 16 + s
    base = s * RPT
    pltpu.async_copy(dst_hbm.at[wid], dsts_v, sem_i)
    pltpu.sync_copy(ones_hbm, ones_v)
    pltpu.sync_copy(zeros_hbm.at[pl.ds(base, RPT)],
                    acc_sh.at[pl.ds(base, RPT)])
    pltpu.make_async_copy(dst_hbm.at[wid], dsts_v, sem_i).wait()
    plsc.subcore_barrier()

    for ci in range(NCHUNK):
        pltpu.async_copy(ones_v, acc_sh.at[dsts_v.at[ci]], sem_s, add=True)
    for ci in range(NCHUNK):
        pltpu.make_async_copy(ones_v, acc_sh.at[dsts_v.at[ci]],
                              sem_s).wait()
    plsc.subcore_barrier()
    pltpu.sync_copy(acc_sh.at[pl.ds(base, RPT)],
                    out_hbm.at[c, pl.ds(base, RPT)])


def _tc_prologue(deg, x_pad, w0):
    def body(deg_ref, x_ref, w_ref, dis_ref, zs_ref):
        d = deg_ref[0, :, 0:1] + deg_ref[1, :, 0:1] + 1.0
        dis = lax.rsqrt(d)
        dis_ref[...] = dis
        zs = jnp.dot(x_ref[...], w_ref[...],
                     preferred_element_type=jnp.float32) * dis
        zs_ref[0] = zs[:, :32]
        zs_ref[1] = zs[:, 32:]

    return pl.pallas_call(
        body,
        out_shape=[jax.ShapeDtypeStruct((NP, 1), jnp.float32),
                   jax.ShapeDtypeStruct((2, NP, 32), jnp.float32)],
    )(deg, x_pad, w0)


def _tc_layer(p, zs, dis, b, w_next, last):
    def body(p_ref, zs_ref, dis_ref, b_ref, w_ref, out_ref):
        dis = dis_ref[...]
        t = jnp.concatenate(
            [p_ref[0, 0] + p_ref[1, 0] + zs_ref[0],
             p_ref[0, 1] + p_ref[1, 1] + zs_ref[1]], axis=1)
        t = t * dis + b_ref[...]
        h = jnp.maximum(t, 0.0)
        z = jnp.dot(h, w_ref[...], preferred_element_type=jnp.float32) * dis
        if last:
            out_ref[0] = z
        else:
            out_ref[0] = z[:, :32]
            out_ref[1] = z[:, 32:]

    d_next = 16 if last else 32
    return pl.pallas_call(
        body,
        out_shape=jax.ShapeDtypeStruct((1 if last else 2, NP, d_next),
                                       jnp.float32),
    )(p, zs, dis, b, w_next)


def _tc_final(p, zs, dis, b):
    def body(p_ref, zs_ref, dis_ref, b_ref, out_ref):
        t = (p_ref[0, 0] + p_ref[1, 0] + zs_ref[0]) * dis_ref[...]
        out_ref[...] = jax.nn.sigmoid(t + b_ref[...])

    return pl.pallas_call(
        body,
        out_shape=jax.ShapeDtypeStruct((NP, 16), jnp.float32),
    )(p, zs, dis, b)


def kernel(x, edge_index, batch,
           W_gcn_0, b_gcn_0, W_gcn_h1, b_gcn_h1, W_gcn_h2, b_gcn_h2,
           W_gcn_h3, b_gcn_h3, W_gcn_h4, b_gcn_h4, W_gcn_h5, b_gcn_h5,
           W_gcn_h6, b_gcn_h6, W_gcn_out, b_gcn_out):
    del batch
    src = edge_index[0].astype(jnp.int32)
    dst = edge_index[1].astype(jnp.int32)
    src2 = jnp.pad(src, (0, EPAD - E)).reshape(NW, NCHUNK, CHUNK)
    dst2 = jnp.pad(dst, (0, EPAD - E),
                   constant_values=NP - 1).reshape(NW, NCHUNK, CHUNK)
    x_pad = jnp.pad(x, ((0, NP - N_NODES), (0, 0)))

    z32 = jnp.zeros((NP, 32), jnp.float32)
    z16 = jnp.zeros((NP, 16), jnp.float32)
    z8 = jnp.zeros((NP, 8), jnp.float32)
    ones_c = jnp.ones((CHUNK, 8), jnp.float32)

    deg = _degree(dst2, ones_c, z8)
    dis, zs = _tc_prologue(deg, x_pad, W_gcn_0)

    ws = [W_gcn_h1, W_gcn_h2, W_gcn_h3, W_gcn_h4, W_gcn_h5, W_gcn_h6]
    bs = [b_gcn_0, b_gcn_h1, b_gcn_h2, b_gcn_h3, b_gcn_h4, b_gcn_h5,
          b_gcn_h6]
    w_out16 = jnp.pad(W_gcn_out, ((0, 0), (0, 10)))
    b_out16 = jnp.pad(b_gcn_out, (0, 10)).reshape(1, 16)

    for i in range(7):
        p = _prop32(zs, src2, dst2, z32)
        if i < 6:
            zs = _tc_layer(p, zs, dis, bs[i].reshape(1, 64), ws[i], False)
        else:
            zs = _tc_layer(p, zs, dis, bs[i].reshape(1, 64), w_out16, True)

    p = _prop16(zs, src2, dst2, z16)
    out = _tc_final(p, zs, dis, b_out16)
    return out[:N_NODES, :6]
